# baseline (speedup 1.0000x reference)
import functools

import jax
import jax.numpy as jnp
from jax import lax
from jax.experimental import pallas as pl
from jax.experimental.pallas import tpu as pltpu
from jax.experimental.pallas import tpu_sc as plsc

D = 64
SCALE = 8.0
_NC, _NS = 2, 16
_NW = _NC * _NS
_V = 1000000
_NBLK = _V // 128
_A_MAIN = _NBLK // _NW
_A_EXTRA = _NBLK - _A_MAIN * _NW
_NJ, _NI = 200, 4096
_NBI = _NI // 128
_B_PER_W = _NJ * _NBI // _NW


def _mesh():
    return plsc.VectorSubcoreMesh(
        core_axis_name="c", subcore_axis_name="s",
        num_cores=_NC, num_subcores=_NS,
    )


def _a_body(tt_hbm, ttail_hbm, s_hbm, vbuf, tmp, pbuf, i0, i1, o0, o1):
    wid = lax.axis_index("s") * _NC + lax.axis_index("c")
    iota = lax.iota(jnp.int32, 16)
    isem = [i0, i1]
    osem = [o0, o1]
    c0s = [(16 * t) & 63 for t in range(8)]
    shs = [(16 * t) >> 6 for t in range(8)]

    def in_slice(u):
        col = pl.multiple_of(u * 128, 128)
        return tt_hbm.at[:, pl.ds(col, 128)]

    def out_slice(u):
        row = pl.multiple_of(u * 64, 64)
        return s_hbm.at[pl.ds(row, 64)]

    def compute(p):
        @pl.loop(0, D, unroll=4)
        def _d(d):
            dv = iota * 0 + d
            for k in range(8):
                val = vbuf[p, d, pl.ds(16 * k, 16)]
                plsc.store_scatter(tmp, [16 * k + iota, dv], val)

        @pl.loop(0, D, unroll=4)
        def _q(q):
            for t in range(8):
                val = tmp[2 * q + shs[t], pl.ds(c0s[t], 16)]
                pbuf[p, q, pl.ds(16 * t, 16)] = val

    base = wid * _A_MAIN
    pltpu.async_copy(in_slice(base), vbuf.at[0], isem[0])

    @pl.loop(0, _A_MAIN, step=2)
    def _t0(t0):
        for p in range(2):
            t = t0 + p
            u = base + t

            @pl.when(t + 1 < _A_MAIN)
            def _():
                pltpu.async_copy(in_slice(u + 1), vbuf.at[1 - p], isem[1 - p])

            pltpu.make_async_copy(in_slice(u), vbuf.at[p], isem[p]).wait()

            @pl.when(t >= 2)
            def _():
                pltpu.make_async_copy(pbuf.at[p], out_slice(u), osem[p]).wait()

            compute(p)
            pltpu.async_copy(pbuf.at[p], out_slice(u), osem[p])

    for p in range(2):
        pltpu.make_async_copy(
            pbuf.at[p], out_slice(base + _A_MAIN - 2 + p), osem[p]
        ).wait()

    @pl.when(wid < _A_EXTRA)
    def _():
        u = _A_MAIN * _NW + wid
        pltpu.sync_copy(in_slice(u), vbuf.at[0])
        compute(0)
        pltpu.sync_copy(pbuf.at[0], out_slice(u))

    @pl.when(wid == _A_EXTRA)
    def _():
        pltpu.sync_copy(ttail_hbm, vbuf.at[0])
        compute(0)
        row = pl.multiple_of((_V - 128) // 2, 8)
        pltpu.sync_copy(pbuf.at[0], s_hbm.at[pl.ds(row, D)])


def _b_body(xtf_hbm, s_hbm, out_hbm, idxbig, rows, obuf,
            g0, g1, g2, g3, o0, o1):
    wid = lax.axis_index("s") * _NC + lax.axis_index("c")
    iota = lax.iota(jnp.int32, 16)
    gsem = [g0, g1, g2, g3]
    osem = [o0, o1]
    avs = [(16 * g + iota) >> 3 for g in range(4)]
    dvs = [(16 * g + iota) & 7 for g in range(4)]

    base = wid * _B_PER_W
    pltpu.sync_copy(xtf_hbm.at[pl.ds(base, _B_PER_W)], idxbig)

    def gather(t, p):
        return pltpu.async_copy(s_hbm.at[idxbig.at[t]], rows.at[p], gsem[p])

    def out_copies(u, p, start):
        j = u >> 5
        b = u & (_NBI - 1)
        for a in range(8):
            src = obuf.at[p, a, :, pl.ds(0, 128)]
            dst = out_hbm.at[j, a, b]
            if start:
                pltpu.async_copy(src, dst, osem[p])
            else:
                pltpu.make_async_copy(src, dst, osem[p]).wait()

    def compute(p, op):
        for i in range(128):
            iv = iota * 0 + i
            for g in range(4):
                val = rows[p, i, pl.ds(16 * g, 16)] * SCALE
                plsc.store_scatter(obuf.at[op], [avs[g], dvs[g], iv], val)

    for p in range(3):
        gather(p, p)

    @pl.loop(0, _B_PER_W, step=4)
    def _t0(t0):
        for p in range(4):
            t = t0 + p
            u = base + t
            op = p & 1

            @pl.when(t + 3 < _B_PER_W)
            def _():
                gather(t + 3, (p + 3) & 3)

            pltpu.make_async_copy(
                s_hbm.at[idxbig.at[t]], rows.at[p], gsem[p]
            ).wait()

            @pl.when(t >= 2)
            def _():
                out_copies(u - 2, op, start=False)

            compute(p, op)
            out_copies(u, op, start=True)

    for p in range(2):
        out_copies(base + _B_PER_W - 2 + p, p, start=False)


@jax.jit
def _run(x, table):
    tt = table.T
    a = pl.kernel(
        _a_body,
        out_type=jax.ShapeDtypeStruct((_V // 2, 128), jnp.float32),
        mesh=_mesh(),
        scratch_types=[
            pltpu.VMEM((2, D, 128), jnp.float32),
            pltpu.VMEM((128, 67), jnp.float32),
            pltpu.VMEM((2, D, 128), jnp.float32),
            pltpu.SemaphoreType.DMA,
            pltpu.SemaphoreType.DMA,
            pltpu.SemaphoreType.DMA,
            pltpu.SemaphoreType.DMA,
        ],
        compiler_params=pltpu.CompilerParams(needs_layout_passes=False),
    )
    ttail = table[_V - 128:].T
    s = a(tt, ttail)
    s2 = s.reshape(_V, D)
    xtf = x.T.reshape(_NJ * _NI // 128, 128)
    b = pl.kernel(
        _b_body,
        out_type=jax.ShapeDtypeStruct((_NJ, 8, _NBI, 8, 128), jnp.float32),
        mesh=_mesh(),
        scratch_types=[
            pltpu.VMEM((_B_PER_W, 128), jnp.int32),
            pltpu.VMEM((4, 128, D), jnp.float32),
            pltpu.VMEM((2, 8, 8, 133), jnp.float32),
            pltpu.SemaphoreType.DMA,
            pltpu.SemaphoreType.DMA,
            pltpu.SemaphoreType.DMA,
            pltpu.SemaphoreType.DMA,
            pltpu.SemaphoreType.DMA,
            pltpu.SemaphoreType.DMA,
        ],
        compiler_params=pltpu.CompilerParams(
            use_tc_tiling_on_sc=False, needs_layout_passes=False
        ),
    )
    out4 = b(xtf, s2)
    return out4.transpose(2, 4, 0, 1, 3).reshape(_NI, _NJ, D)


def kernel(x, table):
    return _run(x, table)


# speedup vs baseline: 2.2320x; 2.2320x over previous
import functools

import jax
import jax.numpy as jnp
from jax import lax
from jax.experimental import pallas as pl
from jax.experimental.pallas import tpu as pltpu
from jax.experimental.pallas import tpu_sc as plsc

D = 64
SCALE = 8.0
_NC, _NS = 2, 16
_NW = _NC * _NS
_V = 1000000
_NBLK = _V // 128
_A_MAIN = _NBLK // _NW
_A_EXTRA = _NBLK - _A_MAIN * _NW
_NJ, _NI = 200, 4096
_NBI = _NI // 128
_B_PER_W = _NJ * _NBI // _NW


def _mesh():
    return plsc.VectorSubcoreMesh(
        core_axis_name="c", subcore_axis_name="s",
        num_cores=_NC, num_subcores=_NS,
    )


def _a_body(tt_hbm, ttail_hbm, s_hbm, vbuf, tmp, pbuf, i0, i1, o0, o1):
    wid = lax.axis_index("s") * _NC + lax.axis_index("c")
    iota = lax.iota(jnp.int32, 16)
    isem = [i0, i1]
    osem = [o0, o1]
    c0s = [(16 * t) & 63 for t in range(8)]
    shs = [(16 * t) >> 6 for t in range(8)]

    def in_slice(u):
        col = pl.multiple_of(u * 128, 128)
        return tt_hbm.at[:, pl.ds(col, 128)]

    def out_slice(u):
        row = pl.multiple_of(u * 64, 64)
        return s_hbm.at[pl.ds(row, 64)]

    def compute(p):
        @plsc.parallel_loop(0, D, unroll=4)
        def _d(d):
            dv = iota * 0 + d
            for k in range(8):
                val = vbuf[p, d, pl.ds(16 * k, 16)]
                plsc.store_scatter(tmp, [16 * k + iota, dv], val)

        @plsc.parallel_loop(0, D, unroll=4)
        def _q(q):
            for t in range(8):
                val = tmp[2 * q + shs[t], pl.ds(c0s[t], 16)]
                pbuf[p, q, pl.ds(16 * t, 16)] = val

    base = wid * _A_MAIN
    pltpu.async_copy(in_slice(base), vbuf.at[0], isem[0])

    @pl.loop(0, _A_MAIN, step=2)
    def _t0(t0):
        for p in range(2):
            t = t0 + p
            u = base + t

            @pl.when(t + 1 < _A_MAIN)
            def _():
                pltpu.async_copy(in_slice(u + 1), vbuf.at[1 - p], isem[1 - p])

            pltpu.make_async_copy(in_slice(u), vbuf.at[p], isem[p]).wait()

            @pl.when(t >= 2)
            def _():
                pltpu.make_async_copy(pbuf.at[p], out_slice(u), osem[p]).wait()

            compute(p)
            pltpu.async_copy(pbuf.at[p], out_slice(u), osem[p])

    for p in range(2):
        pltpu.make_async_copy(
            pbuf.at[p], out_slice(base + _A_MAIN - 2 + p), osem[p]
        ).wait()

    @pl.when(wid < _A_EXTRA)
    def _():
        u = _A_MAIN * _NW + wid
        pltpu.sync_copy(in_slice(u), vbuf.at[0])
        compute(0)
        pltpu.sync_copy(pbuf.at[0], out_slice(u))

    @pl.when(wid == _A_EXTRA)
    def _():
        pltpu.sync_copy(ttail_hbm, vbuf.at[0])
        compute(0)
        row = pl.multiple_of((_V - 128) // 2, 8)
        pltpu.sync_copy(pbuf.at[0], s_hbm.at[pl.ds(row, D)])


def _b_body(xtf_hbm, s_hbm, out_hbm, idxbig, rows, obuf,
            g0, g1, g2, g3, o0, o1):
    wid = lax.axis_index("s") * _NC + lax.axis_index("c")
    iota = lax.iota(jnp.int32, 16)
    gsem = [g0, g1, g2, g3]
    osem = [o0, o1]
    avs = [(16 * g + iota) >> 3 for g in range(4)]
    dvs = [(16 * g + iota) & 7 for g in range(4)]

    base = wid * _B_PER_W
    pltpu.sync_copy(xtf_hbm.at[pl.ds(base, _B_PER_W)], idxbig)

    def gather(t, p):
        return pltpu.async_copy(s_hbm.at[idxbig.at[t]], rows.at[p], gsem[p])

    def out_copies(u, p, start):
        j = u >> 5
        b = u & (_NBI - 1)
        for a in range(8):
            src = obuf.at[p, a, :, pl.ds(0, 128)]
            dst = out_hbm.at[j, a, b]
            if start:
                pltpu.async_copy(src, dst, osem[p])
            else:
                pltpu.make_async_copy(src, dst, osem[p]).wait()

    def compute(p, op):
        @plsc.parallel_loop(0, 128, unroll=4)
        def _i(i):
            iv = iota * 0 + i
            for g in range(4):
                val = rows[p, i, pl.ds(16 * g, 16)] * SCALE
                plsc.store_scatter(obuf.at[op], [avs[g], dvs[g], iv], val)

    for p in range(3):
        gather(p, p)

    @pl.loop(0, _B_PER_W, step=4)
    def _t0(t0):
        for p in range(4):
            t = t0 + p
            u = base + t
            op = p & 1

            @pl.when(t + 3 < _B_PER_W)
            def _():
                gather(t + 3, (p + 3) & 3)

            pltpu.make_async_copy(
                s_hbm.at[idxbig.at[t]], rows.at[p], gsem[p]
            ).wait()

            @pl.when(t >= 2)
            def _():
                out_copies(u - 2, op, start=False)

            compute(p, op)
            out_copies(u, op, start=True)

    for p in range(2):
        out_copies(base + _B_PER_W - 2 + p, p, start=False)


@jax.jit
def _run(x, table):
    tt = table.T
    a = pl.kernel(
        _a_body,
        out_type=jax.ShapeDtypeStruct((_V // 2, 128), jnp.float32),
        mesh=_mesh(),
        scratch_types=[
            pltpu.VMEM((2, D, 128), jnp.float32),
            pltpu.VMEM((128, 67), jnp.float32),
            pltpu.VMEM((2, D, 128), jnp.float32),
            pltpu.SemaphoreType.DMA,
            pltpu.SemaphoreType.DMA,
            pltpu.SemaphoreType.DMA,
            pltpu.SemaphoreType.DMA,
        ],
        compiler_params=pltpu.CompilerParams(needs_layout_passes=False),
    )
    ttail = table[_V - 128:].T
    s = a(tt, ttail)
    s2 = s.reshape(_V, D)
    xtf = x.T.reshape(_NJ * _NI // 128, 128)
    b = pl.kernel(
        _b_body,
        out_type=jax.ShapeDtypeStruct((_NJ, 8, _NBI, 8, 128), jnp.float32),
        mesh=_mesh(),
        scratch_types=[
            pltpu.VMEM((_B_PER_W, 128), jnp.int32),
            pltpu.VMEM((4, 128, D), jnp.float32),
            pltpu.VMEM((2, 8, 8, 133), jnp.float32),
            pltpu.SemaphoreType.DMA,
            pltpu.SemaphoreType.DMA,
            pltpu.SemaphoreType.DMA,
            pltpu.SemaphoreType.DMA,
            pltpu.SemaphoreType.DMA,
            pltpu.SemaphoreType.DMA,
        ],
        compiler_params=pltpu.CompilerParams(
            use_tc_tiling_on_sc=False, needs_layout_passes=False
        ),
    )
    out4 = b(xtf, s2)
    return out4.transpose(2, 4, 0, 1, 3).reshape(_NI, _NJ, D)


def kernel(x, table):
    return _run(x, table)
